# Initial kernel scaffold; baseline (speedup 1.0000x reference)
import jax
import jax.numpy as jnp
from jax.experimental import pallas as pl


def kernel(x, protos):
    raise NotImplementedError("write your pallas kernel here")


# speedup vs baseline: 2.0640x; 2.0640x over previous
import functools

import jax
import jax.numpy as jnp
from jax.experimental import pallas as pl


def _reduce_body(x_ref, o_ref, *, scale):
    o_ref[...] = jnp.sum(x_ref[...], axis=(1, 3), keepdims=True)[:, :, :, 0] * scale


def kernel(x, protos):
    N, M, C, T, V = x.shape
    P = protos.shape[0]
    scale = 1.0 / (M * P)
    xr = x.reshape(N, M, C, T * V)

    out = pl.pallas_call(
        functools.partial(_reduce_body, scale=scale),
        grid=(N,),
        in_specs=[pl.BlockSpec((1, M, C, T * V), lambda n: (n, 0, 0, 0))],
        out_specs=pl.BlockSpec((1, 1, C), lambda n: (n, 0, 0)),
        out_shape=jax.ShapeDtypeStruct((N, 1, C), x.dtype),
    )(xr)
    return out.reshape(N, C)
